# Initial kernel scaffold; baseline (speedup 1.0000x reference)
import jax
import jax.numpy as jnp
from jax.experimental import pallas as pl


def kernel(x, adj, src, tgt, Msrc, Mtgt, Wf, bf, Ww, bw):
    raise NotImplementedError("write your pallas kernel here")


# speedup vs baseline: 1.3023x; 1.3023x over previous
import functools

import jax
import jax.numpy as jnp
from jax import lax
from jax.experimental import pallas as pl
from jax.experimental.pallas import tpu as pltpu

N, E, DI, DO = 4096, 32768, 128, 128
EPS = 1e-06
DP = 144
EB = 512


def _precompute_body(x_ref, w1_ref, w2_ref, b_ref, pu_ref, qv_ref):
    xb = x_ref[...]
    pu_ref[...] = jnp.dot(xb, w1_ref[...], preferred_element_type=jnp.float32)
    qv_ref[...] = (
        jnp.dot(xb, w2_ref[...], preferred_element_type=jnp.float32) + b_ref[...]
    )


def _agg_body(mtgt_ref, hs_ref, ht_ref, o_ref, acc_ref):
    i = pl.program_id(0)

    @pl.when(i == 0)
    def _init():
        acc_ref[...] = jnp.zeros_like(acc_ref)

    full = hs_ref[...] + ht_ref[...]
    y = jnp.maximum(full[:, :DO], 0.0)
    a = full[:, DO:DO + 1]
    z = jnp.concatenate([y * a, full[:, DO:]], axis=1)
    acc_ref[...] += jnp.dot(mtgt_ref[...], z, preferred_element_type=jnp.float32)

    @pl.when(i == pl.num_programs(0) - 1)
    def _fini():
        o_ref[...] = acc_ref[:, :DO] / (acc_ref[:, DO:DO + 1] + EPS)


@jax.jit
def _run(x, src, tgt, Mtgt, W1, W2, bvec):
    pu, qv = pl.pallas_call(
        _precompute_body,
        out_shape=[
            jax.ShapeDtypeStruct((N, DP), jnp.float32),
            jax.ShapeDtypeStruct((N, DP), jnp.float32),
        ],
    )(x, W1, W2, bvec)

    hs = pu[src]
    ht = qv[tgt]

    o = pl.pallas_call(
        _agg_body,
        grid=(E // EB,),
        in_specs=[
            pl.BlockSpec((N, EB), lambda i: (0, i)),
            pl.BlockSpec((EB, DP), lambda i: (i, 0)),
            pl.BlockSpec((EB, DP), lambda i: (i, 0)),
        ],
        out_specs=pl.BlockSpec((N, DO), lambda i: (0, 0)),
        out_shape=jax.ShapeDtypeStruct((N, DO), jnp.float32),
        scratch_shapes=[pltpu.VMEM((N, DP), jnp.float32)],
    )(Mtgt, hs, ht)
    return o


def kernel(x, adj, src, tgt, Msrc, Mtgt, Wf, bf, Ww, bw):
    src = src.astype(jnp.int32)
    tgt = tgt.astype(jnp.int32)
    pad = jnp.zeros((DI, DP - DO - 1), jnp.float32)
    W1 = jnp.concatenate([Wf[:, :DI].T, Ww[:, :DI].T, pad], axis=1)
    W2 = jnp.concatenate([Wf[:, DI:].T, Ww[:, DI:].T, pad], axis=1)
    bvec = jnp.concatenate(
        [bf, bw, jnp.zeros((DP - DO - 1,), jnp.float32)]
    ).reshape(1, DP)
    return _run(x, src, tgt, Mtgt, W1, W2, bvec)


# speedup vs baseline: 2.4458x; 1.8780x over previous
import functools

import jax
import jax.numpy as jnp
from jax import lax
from jax.experimental import pallas as pl
from jax.experimental.pallas import tpu as pltpu
from jax.experimental.pallas import tpu_sc as plsc

N, E, DI, DO = 4096, 32768, 128, 128
EPS = 1e-06
EB = 512
CH = 128
AW = 8
DZ = DO + AW


def _make_gather(nc, ns):
    nw = nc * ns
    epw = E // nw
    chw = epw // CH

    def body(x_hbm, src_hbm, tgt_hbm, xs_hbm, xt_hbm,
             sidx, tidx, buf_a, buf_b, sem_a, sem_b):
        wid = lax.axis_index("s") * nc + lax.axis_index("c")
        base = wid * epw
        pltpu.sync_copy(src_hbm.at[pl.ds(wid * chw, chw)], sidx)
        pltpu.sync_copy(tgt_hbm.at[pl.ds(wid * chw, chw)], tidx)
        for k in range(chw):
            ca = pltpu.async_copy(x_hbm.at[sidx.at[k]], buf_a, sem_a)
            cb = pltpu.async_copy(x_hbm.at[tidx.at[k]], buf_b, sem_b)
            ca.wait()
            pltpu.sync_copy(buf_a, xs_hbm.at[pl.ds(base + k * CH, CH)])
            cb.wait()
            pltpu.sync_copy(buf_b, xt_hbm.at[pl.ds(base + k * CH, CH)])

    return pl.kernel(
        body,
        out_type=[
            jax.ShapeDtypeStruct((E, DI), jnp.float32),
            jax.ShapeDtypeStruct((E, DI), jnp.float32),
        ],
        mesh=plsc.VectorSubcoreMesh(core_axis_name="c", subcore_axis_name="s"),
        scratch_types=[
            pltpu.VMEM((chw, CH), jnp.int32),
            pltpu.VMEM((chw, CH), jnp.int32),
            pltpu.VMEM((CH, DI), jnp.float32),
            pltpu.VMEM((CH, DI), jnp.float32),
            pltpu.SemaphoreType.DMA,
            pltpu.SemaphoreType.DMA,
        ],
    )


def _agg_body(mtgt_ref, xs_ref, xt_ref, w1_ref, w2_ref, b_ref, o_ref, acc_ref):
    i = pl.program_id(0)

    @pl.when(i == 0)
    def _init():
        acc_ref[...] = jnp.zeros_like(acc_ref)

    f = (jnp.dot(xs_ref[...], w1_ref[...], preferred_element_type=jnp.float32)
         + jnp.dot(xt_ref[...], w2_ref[...], preferred_element_type=jnp.float32)
         + b_ref[...])
    y = jnp.maximum(f[:, :DO], 0.0)
    a = f[:, DO:DO + 1]
    z = jnp.concatenate([y * a, f[:, DO:]], axis=1)
    acc_ref[...] += jnp.dot(mtgt_ref[...], z, preferred_element_type=jnp.float32)

    @pl.when(i == pl.num_programs(0) - 1)
    def _fini():
        o_ref[...] = acc_ref[:, :DO] / (acc_ref[:, DO:DO + 1] + EPS)


@functools.partial(jax.jit, static_argnames=("nc", "ns"))
def _run(x, src2d, tgt2d, Mtgt, W1a, W2a, bvec, nc, ns):
    xs, xt = _make_gather(nc, ns)(x, src2d, tgt2d)

    o = pl.pallas_call(
        _agg_body,
        grid=(E // EB,),
        in_specs=[
            pl.BlockSpec((N, EB), lambda i: (0, i)),
            pl.BlockSpec((EB, DI), lambda i: (i, 0)),
            pl.BlockSpec((EB, DI), lambda i: (i, 0)),
            pl.BlockSpec((DI, DZ), lambda i: (0, 0)),
            pl.BlockSpec((DI, DZ), lambda i: (0, 0)),
            pl.BlockSpec((1, DZ), lambda i: (0, 0)),
        ],
        out_specs=pl.BlockSpec((N, DO), lambda i: (0, 0)),
        out_shape=jax.ShapeDtypeStruct((N, DO), jnp.float32),
        scratch_shapes=[pltpu.VMEM((N, DZ), jnp.float32)],
    )(Mtgt, xs, xt, W1a, W2a, bvec)
    return o


def kernel(x, adj, src, tgt, Msrc, Mtgt, Wf, bf, Ww, bw):
    src2d = src.astype(jnp.int32).reshape(E // CH, CH)
    tgt2d = tgt.astype(jnp.int32).reshape(E // CH, CH)
    zpad = jnp.zeros((DI, AW - 1), jnp.float32)
    W1a = jnp.concatenate([Wf[:, :DI].T, Ww[:, :DI].T, zpad], axis=1)
    W2a = jnp.concatenate([Wf[:, DI:].T, Ww[:, DI:].T, zpad], axis=1)
    bvec = jnp.concatenate(
        [bf, bw, jnp.zeros((AW - 1,), jnp.float32)]
    ).reshape(1, DZ)
    info = plsc.get_sparse_core_info()
    return _run(x, src2d, tgt2d, Mtgt, W1a, W2a, bvec,
                nc=info.num_cores, ns=info.num_subcores)


# speedup vs baseline: 2.5462x; 1.0411x over previous
import functools

import jax
import jax.numpy as jnp
from jax import lax
from jax.experimental import pallas as pl
from jax.experimental.pallas import tpu as pltpu
from jax.experimental.pallas import tpu_sc as plsc

N, E, DI, DO = 4096, 32768, 128, 128
EPS = 1e-06
EB = 1024
CH = 128
AW = 8
DZ = DO + AW


def _make_gather(nc, ns):
    nw = nc * ns
    epw = E // nw
    chw = epw // CH

    def body(x_hbm, src_hbm, tgt_hbm, xs_hbm, xt_hbm,
             sidx, tidx, buf_a, buf_b, sem_a, sem_b):
        wid = lax.axis_index("s") * nc + lax.axis_index("c")
        base = wid * epw
        pltpu.sync_copy(src_hbm.at[pl.ds(wid * chw, chw)], sidx)
        pltpu.sync_copy(tgt_hbm.at[pl.ds(wid * chw, chw)], tidx)
        for k in range(chw):
            ca = pltpu.async_copy(x_hbm.at[sidx.at[k]], buf_a, sem_a)
            cb = pltpu.async_copy(x_hbm.at[tidx.at[k]], buf_b, sem_b)
            ca.wait()
            pltpu.sync_copy(buf_a, xs_hbm.at[pl.ds(base + k * CH, CH)])
            cb.wait()
            pltpu.sync_copy(buf_b, xt_hbm.at[pl.ds(base + k * CH, CH)])

    return pl.kernel(
        body,
        out_type=[
            jax.ShapeDtypeStruct((E, DI), jnp.float32),
            jax.ShapeDtypeStruct((E, DI), jnp.float32),
        ],
        mesh=plsc.VectorSubcoreMesh(core_axis_name="c", subcore_axis_name="s"),
        scratch_types=[
            pltpu.VMEM((chw, CH), jnp.int32),
            pltpu.VMEM((chw, CH), jnp.int32),
            pltpu.VMEM((CH, DI), jnp.float32),
            pltpu.VMEM((CH, DI), jnp.float32),
            pltpu.SemaphoreType.DMA,
            pltpu.SemaphoreType.DMA,
        ],
    )


def _agg_body(mtgt_ref, xs_ref, xt_ref, w1_ref, w2_ref, b_ref, o_ref, acc_ref):
    i = pl.program_id(0)

    @pl.when(i == 0)
    def _init():
        acc_ref[...] = jnp.zeros_like(acc_ref)

    f = (jnp.dot(xs_ref[...], w1_ref[...], preferred_element_type=jnp.float32)
         + jnp.dot(xt_ref[...], w2_ref[...], preferred_element_type=jnp.float32)
         + b_ref[...])
    y = jnp.maximum(f[:, :DO], 0.0)
    a = f[:, DO:DO + 1]
    z = jnp.concatenate([y * a, f[:, DO:]], axis=1)
    acc_ref[...] += jnp.dot(mtgt_ref[...], z, preferred_element_type=jnp.float32)

    @pl.when(i == pl.num_programs(0) - 1)
    def _fini():
        o_ref[...] = acc_ref[:, :DO] / (acc_ref[:, DO:DO + 1] + EPS)


@functools.partial(jax.jit, static_argnames=("nc", "ns"))
def _run(x, src2d, tgt2d, Mtgt, W1a, W2a, bvec, nc, ns):
    xs, xt = _make_gather(nc, ns)(x, src2d, tgt2d)

    o = pl.pallas_call(
        _agg_body,
        grid=(E // EB,),
        in_specs=[
            pl.BlockSpec((N, EB), lambda i: (0, i)),
            pl.BlockSpec((EB, DI), lambda i: (i, 0)),
            pl.BlockSpec((EB, DI), lambda i: (i, 0)),
            pl.BlockSpec((DI, DZ), lambda i: (0, 0)),
            pl.BlockSpec((DI, DZ), lambda i: (0, 0)),
            pl.BlockSpec((1, DZ), lambda i: (0, 0)),
        ],
        out_specs=pl.BlockSpec((N, DO), lambda i: (0, 0)),
        out_shape=jax.ShapeDtypeStruct((N, DO), jnp.float32),
        scratch_shapes=[pltpu.VMEM((N, DZ), jnp.float32)],
    )(Mtgt, xs, xt, W1a, W2a, bvec)
    return o


def kernel(x, adj, src, tgt, Msrc, Mtgt, Wf, bf, Ww, bw):
    src2d = src.astype(jnp.int32).reshape(E // CH, CH)
    tgt2d = tgt.astype(jnp.int32).reshape(E // CH, CH)
    zpad = jnp.zeros((DI, AW - 1), jnp.float32)
    W1a = jnp.concatenate([Wf[:, :DI].T, Ww[:, :DI].T, zpad], axis=1)
    W2a = jnp.concatenate([Wf[:, DI:].T, Ww[:, DI:].T, zpad], axis=1)
    bvec = jnp.concatenate(
        [bf, bw, jnp.zeros((AW - 1,), jnp.float32)]
    ).reshape(1, DZ)
    info = plsc.get_sparse_core_info()
    return _run(x, src2d, tgt2d, Mtgt, W1a, W2a, bvec,
                nc=info.num_cores, ns=info.num_subcores)
